# baseline (speedup 1.0000x reference)
import jax
import jax.numpy as jnp
from jax.experimental import pallas as pl
from jax.experimental.pallas import tpu as pltpu


def _copy_kernel(in_ref, out_ref, sem):
    pltpu.make_async_copy(in_ref, out_ref, sem).start()
    pltpu.make_async_copy(in_ref, out_ref, sem).wait()


def kernel(stems):
    shape = stems.shape
    flat = stems.reshape(-1, shape[-1])
    out = pl.pallas_call(
        _copy_kernel,
        out_shape=jax.ShapeDtypeStruct(flat.shape, flat.dtype),
        in_specs=[pl.BlockSpec(memory_space=pltpu.MemorySpace.HBM)],
        out_specs=pl.BlockSpec(memory_space=pltpu.MemorySpace.HBM),
        scratch_shapes=[pltpu.SemaphoreType.DMA],
    )(flat)
    return out.reshape(shape)


# speedup vs baseline: 8.1379x; 8.1379x over previous
import jax
import jax.numpy as jnp
from jax.experimental import pallas as pl
from jax.experimental.pallas import tpu as pltpu


_ROWS = 128
_COLS = 131072
_BLOCK_ROWS = 8


def _copy_kernel(in_ref, out_ref):
    out_ref[...] = in_ref[...]


def kernel(stems):
    shape = stems.shape
    flat = stems.reshape(_ROWS, _COLS)
    out = pl.pallas_call(
        _copy_kernel,
        out_shape=jax.ShapeDtypeStruct(flat.shape, flat.dtype),
        grid=(_ROWS // _BLOCK_ROWS,),
        in_specs=[pl.BlockSpec((_BLOCK_ROWS, _COLS), lambda i: (i, 0))],
        out_specs=pl.BlockSpec((_BLOCK_ROWS, _COLS), lambda i: (i, 0)),
    )(flat)
    return out.reshape(shape)
